# baseline (speedup 1.0000x reference)
import jax
import jax.numpy as jnp
from jax import lax
from jax.experimental import pallas as pl
from jax.experimental.pallas import tpu as pltpu
from jax.experimental.pallas import tpu_sc as plsc

NC, NS = 2, 16
NW = NC * NS
BB = 128
DIM = 64


def _sc_body(idx4_hbm, tok_hbm, pos_hbm, out_hbm,
             idx_v, rows0, rows1, rows2, rows3, tb0, tb1, pos_v, skew_v,
             gsem0, gsem1, gsem2, gsem3, osem0, osem1):
    seq = idx4_hbm.shape[0] * idx4_hbm.shape[2]
    wid = lax.axis_index("s") * NC + lax.axis_index("c")

    rows = (rows0, rows1, rows2, rows3)
    tbs = (tb0, tb1)
    gsems = (gsem0, gsem1, gsem2, gsem3)
    osems = (osem0, osem1)

    pltpu.sync_copy(pos_hbm, pos_v)
    pltpu.sync_copy(idx4_hbm.at[:, pl.ds(wid, 1)], idx_v)

    iota = lax.iota(jnp.int32, 16)
    zero16 = jnp.zeros((16,), jnp.int32)

    @plsc.parallel_loop(0, 16, 1)
    def _(t):
        m = (t + iota) % 16
        skew_v[0, t, pl.ds(0, 16)] = m
        skew_v[1, t, pl.ds(0, 16)] = m // 8
        skew_v[2, t, pl.ds(0, 16)] = (m % 8) * 128 + iota

    def fire_gather(s, b):
        pltpu.async_copy(tok_hbm.at[idx_v.at[s // 8, 0, s % 8]], rows[b], gsems[b])

    def drain_gather(b):
        pltpu.make_async_copy(tok_hbm.at[pl.ds(0, BB)], rows[b], gsems[b]).wait()

    def fire_out(s, b):
        pltpu.async_copy(tbs[b], out_hbm.at[s, :, pl.ds(wid, 1)], osems[b])

    def drain_out(b):
        pltpu.make_async_copy(tbs[b], out_hbm.at[0, :, pl.ds(0, 1)], osems[b]).wait()

    def compute(s, b4, b2):
        rb, tb = rows[b4], tbs[b2]
        s_splat = jnp.full((16,), s, jnp.int32)
        cvecs = [iota + 16 * j for j in range(BB // 16)]

        @plsc.parallel_loop(0, 16, 1)
        def _(t):
            sl = pl.ds(0, 16)
            m = skew_v[0, t, sl]
            kb = skew_v[1, t, sl]
            wb = skew_v[2, t, sl]
            for u in range(DIM // 16):
                dvec = 16 * u + m
                pg = plsc.load_gather(pos_v, [s_splat, dvec])
                kvec = 2 * u + kb
                for j in range(BB // 16):
                    g = plsc.load_gather(rb, [cvecs[j], dvec])
                    plsc.store_scatter(tb, [kvec, zero16, 16 * j + wb],
                                       g * 8.0 + pg)

    for q in range(4):
        fire_gather(q, q)

    def step(s, b4, b2):
        @pl.when(s >= 2)
        def _():
            drain_out(b2)

        drain_gather(b4)
        compute(s, b4, b2)
        fire_out(s, b2)

        @pl.when(s + 4 < seq)
        def _():
            fire_gather(s + 4, b4)

    def quad(t, _):
        for q in range(4):
            step(4 * t + q, q, q % 2)
        return _

    lax.fori_loop(0, seq // 4, quad, 0)
    drain_out(0)
    drain_out(1)


def kernel(inputs, token_table, position_table):
    batch, seq = inputs.shape
    vocab, dim = token_table.shape
    idx4 = (
        inputs.astype(jnp.int32)
        .reshape(batch // BB, BB, seq // 8, 8)
        .transpose(2, 0, 3, 1)
    )

    mesh = plsc.VectorSubcoreMesh(
        core_axis_name="c", subcore_axis_name="s", num_cores=NC, num_subcores=NS
    )
    call = pl.kernel(
        _sc_body,
        out_type=jax.ShapeDtypeStruct((seq, dim // 8, batch // BB, 8 * BB), jnp.float32),
        name="emb_kernel",
        mesh=mesh,
        scratch_types=[
            pltpu.VMEM((seq // 8, 1, 8, BB), jnp.int32),
            pltpu.VMEM((BB, dim), jnp.float32),
            pltpu.VMEM((BB, dim), jnp.float32),
            pltpu.VMEM((BB, dim), jnp.float32),
            pltpu.VMEM((BB, dim), jnp.float32),
            pltpu.VMEM((dim // 8, 1, 8 * BB), jnp.float32),
            pltpu.VMEM((dim // 8, 1, 8 * BB), jnp.float32),
            pltpu.VMEM((seq, dim), jnp.float32),
            pltpu.VMEM((3, 16, 16), jnp.int32),
            pltpu.SemaphoreType.DMA,
            pltpu.SemaphoreType.DMA,
            pltpu.SemaphoreType.DMA,
            pltpu.SemaphoreType.DMA,
            pltpu.SemaphoreType.DMA,
            pltpu.SemaphoreType.DMA,
        ],
        compiler_params=pltpu.CompilerParams(
            use_tc_tiling_on_sc=False, needs_layout_passes=False
        ),
    )
    a = call(idx4, token_table, position_table)
    return (
        a.reshape(seq, dim // 8, batch // BB, 8, BB)
        .transpose(2, 4, 0, 1, 3)
        .reshape(batch, seq, dim)
    )
